# baseline (speedup 1.0000x reference)
import functools

import jax
import jax.numpy as jnp
from jax import lax
from jax.experimental import pallas as pl
from jax.experimental.pallas import tpu as pltpu
from jax.experimental.pallas import tpu_sc as plsc

N = 32768
D = 256
B = 16
L = 16
NC, NS = 2, 16
NW = NC * NS
BLK = 64
NBLK = N // BLK
SC_ROWS = N // 2
TC_ROWS = N - SC_ROWS
BLK_PER_W = SC_ROWS // BLK // NW
ROWS_PER_W = SC_ROWS // NW
BLK2 = 512
DH = D // 2
EDGE = 2 * BLK + 8
CHUNK = 2 * BLK

_mesh = plsc.VectorSubcoreMesh(core_axis_name="c", subcore_axis_name="s")


def _i32(v):
    return jnp.asarray(v, jnp.int32)


def _lane_i32(vec, m):
    mask = (lax.iota(jnp.int32, L) == m).astype(jnp.int32)
    return jnp.sum(vec * mask, dtype=jnp.int32)


def _lane_f32(vec, m):
    mask = (lax.iota(jnp.int32, L) == m).astype(jnp.float32)
    return jnp.sum(vec * mask, dtype=jnp.float32)


def _tc_body(x_ref, o_ref):
    for j in range(BLK2 // BLK):
        o_ref[pl.ds(j, 1), :] = jnp.sum(
            x_ref[pl.ds(j * BLK, BLK), :], axis=0, keepdims=True)


_tc_block_sums = pl.pallas_call(
    _tc_body,
    grid=(TC_ROWS // BLK2,),
    in_specs=[pl.BlockSpec((BLK2, D),
                           lambda i: (i + SC_ROWS // BLK2, i * 0))],
    out_specs=pl.BlockSpec((BLK2 // BLK, D), lambda i: (i, i * 0)),
    out_shape=jax.ShapeDtypeStruct((TC_ROWS // BLK, D), jnp.float32),
)


@functools.partial(
    pl.kernel,
    out_type=jax.ShapeDtypeStruct((SC_ROWS // BLK, D), jnp.float32),
    mesh=_mesh,
    scratch_types=[
        pltpu.VMEM((2, CHUNK, D), jnp.float32),
        pltpu.VMEM((BLK_PER_W, D), jnp.float32),
        pltpu.SemaphoreType.DMA,
        pltpu.SemaphoreType.DMA,
    ],
)
def _block_sums(x_hbm, bs_hbm, buf, acc_v, sem0, sem1):
    wid = lax.axis_index("s") * NC + lax.axis_index("c")
    row0 = wid * ROWS_PER_W
    sems = (sem0, sem1)
    copies = [None, None]
    CH = CHUNK
    NCH = ROWS_PER_W // CH
    copies[0] = pltpu.async_copy(
        x_hbm.at[pl.ds(row0, CH)], buf.at[_i32(0)], sem0)
    for g in range(NCH):
        cur = g % 2
        if g + 1 < NCH:
            nxt = (g + 1) % 2
            copies[nxt] = pltpu.async_copy(
                x_hbm.at[pl.ds(row0 + (g + 1) * CH, CH)], buf.at[_i32(nxt)],
                sems[nxt])
        copies[cur].wait()
        bb = buf.at[_i32(cur)]
        for sb in range(CH // BLK):

            def body(r, accs, _sb=sb):
                r2 = r + r + _sb * BLK
                r3 = r2 + 1
                accs = tuple(accs[c] + bb[r2, pl.ds(c * L, L)]
                             for c in range(D // L))
                return tuple(accs[c] + bb[r3, pl.ds(c * L, L)]
                             for c in range(D // L))

            accs = lax.fori_loop(
                _i32(0), _i32(BLK // 2), body,
                tuple(jnp.zeros((L,), jnp.float32) for _ in range(D // L)))
            b = g * (CH // BLK) + sb
            for c in range(D // L):
                acc_v[_i32(b), pl.ds(c * L, L)] = accs[c]
    pltpu.sync_copy(acc_v, bs_hbm.at[pl.ds(wid * BLK_PER_W, BLK_PER_W)])


@functools.partial(
    pl.kernel,
    out_type=jax.ShapeDtypeStruct((B * D,), jnp.float32),
    mesh=_mesh,
    scratch_types=[
        pltpu.VMEM((8, L), jnp.int32),
        pltpu.VMEM((L,), jnp.float32),
        pltpu.VMEM((NBLK, DH), jnp.float32),
        pltpu.VMEM((EDGE, DH), jnp.float32),
        pltpu.VMEM((EDGE, DH), jnp.float32),
        pltpu.VMEM((DH,), jnp.float32),
        pltpu.SemaphoreType.DMA,
        pltpu.SemaphoreType.DMA,
        pltpu.SemaphoreType.DMA,
    ],
    compiler_params=pltpu.CompilerParams(needs_layout_passes=False),
)
def _combine(x_hbm, bs_hbm, pi_hbm, inv_hbm, out_hbm,
             pv, invv, bsv, e1v, e2v, outv, sem_bs, sem_e1, sem_e2):
    wid = lax.axis_index("s") * NC + lax.axis_index("c")
    m = wid // 2
    h = wid % 2
    col0 = h * DH

    cp_bs = pltpu.async_copy(
        bs_hbm.at[pl.ds(0, NBLK), pl.ds(col0, DH)], bsv, sem_bs)
    pltpu.sync_copy(pi_hbm, pv)
    pltpu.sync_copy(inv_hbm, invv)
    fb_lo = _lane_i32(pv[_i32(0)], m)
    fb_hi = _lane_i32(pv[_i32(1)], m)
    e1_lo = _lane_i32(pv[_i32(2)], m)
    e1_hi = _lane_i32(pv[_i32(3)], m)
    e2_lo = _lane_i32(pv[_i32(4)], m)
    e2_hi = _lane_i32(pv[_i32(5)], m)
    c1 = pl.multiple_of(_lane_i32(pv[_i32(6)], m), 8)
    c2 = pl.multiple_of(_lane_i32(pv[_i32(7)], m), 8)
    inv = _lane_f32(invv[...], m)

    cp_e1 = pltpu.async_copy(
        x_hbm.at[pl.ds(c1, EDGE), pl.ds(col0, DH)], e1v, sem_e1)
    cp_e2 = pltpu.async_copy(
        x_hbm.at[pl.ds(c2, EDGE), pl.ds(col0, DH)], e2v, sem_e2)

    zero = tuple(jnp.zeros((L,), jnp.float32) for _ in range(DH // L))

    cp_bs.wait()

    def fb_body(bk, accs):
        return tuple(accs[c] + bsv[bk, pl.ds(c * L, L)]
                     for c in range(DH // L))

    accs = lax.fori_loop(fb_lo, fb_hi, fb_body, zero)

    cp_e1.wait()

    def e1_body(r, accs):
        return tuple(accs[c] + e1v[r, pl.ds(c * L, L)]
                     for c in range(DH // L))

    accs = lax.fori_loop(e1_lo - c1, e1_hi - c1, e1_body, accs)

    cp_e2.wait()

    def e2_body(r, accs):
        return tuple(accs[c] + e2v[r, pl.ds(c * L, L)]
                     for c in range(DH // L))

    accs = lax.fori_loop(e2_lo - c2, e2_hi - c2, e2_body, accs)

    for c in range(DH // L):
        outv[pl.ds(c * L, L)] = accs[c] * inv
    pltpu.sync_copy(outv, out_hbm.at[pl.ds(m * D + col0, DH)])


def kernel(atom_hiddens, a_scope):
    x = atom_hiddens.astype(jnp.float32)
    s = a_scope[:, 0].astype(jnp.int32)
    sz = a_scope[:, 1].astype(jnp.int32)
    e = jnp.minimum(s + sz, N)
    b0 = (s + BLK - 1) // BLK
    b1 = e // BLK
    has_full = b0 < b1
    fb_lo = jnp.where(has_full, b0, 0)
    fb_hi = jnp.where(has_full, b1, 0)
    e1_lo = s
    e1_hi = jnp.where(has_full, b0 * BLK, e)
    e2_lo = jnp.where(has_full, b1 * BLK, 0)
    e2_hi = jnp.where(has_full, e, 0)
    c1 = jnp.minimum((e1_lo // 8) * 8, N - EDGE)
    c2 = jnp.minimum((e2_lo // 8) * 8, N - EDGE)
    pi = jnp.stack([fb_lo, fb_hi, e1_lo, e1_hi, e2_lo, e2_hi, c1, c2])
    inv = jnp.where(sz > 0, 1.0 / jnp.maximum(sz, 1).astype(jnp.float32), 0.0)

    bs_sc = _block_sums(x)
    bs_tc = _tc_block_sums(x)
    bs = jnp.concatenate([bs_sc, bs_tc], axis=0)
    return _combine(x, bs, pi, inv).reshape(B, D)


# speedup vs baseline: 1.2189x; 1.2189x over previous
import functools

import jax
import jax.numpy as jnp
from jax import lax
from jax.experimental import pallas as pl
from jax.experimental.pallas import tpu as pltpu
from jax.experimental.pallas import tpu_sc as plsc

N = 32768
D = 256
B = 16
L = 16
NC, NS = 2, 16
NW = NC * NS
BLK = 64
NBLK = N // BLK
SC_ROWS = 8192
TC_ROWS = N - SC_ROWS
SC_NBLK = SC_ROWS // BLK
TC_NBLK = TC_ROWS // BLK
BLK_PER_W = SC_NBLK // NW
ROWS_PER_W = SC_ROWS // NW
BLK2 = 2048
DH = D // 2
EDGE = 2 * BLK + 8
CHUNK = 2 * BLK

_mesh = plsc.VectorSubcoreMesh(core_axis_name="c", subcore_axis_name="s")


def _i32(v):
    return jnp.asarray(v, jnp.int32)


def _lane_i32(vec, m):
    mask = (lax.iota(jnp.int32, L) == m).astype(jnp.int32)
    return jnp.sum(vec * mask, dtype=jnp.int32)


def _lane_f32(vec, m):
    mask = (lax.iota(jnp.int32, L) == m).astype(jnp.float32)
    return jnp.sum(vec * mask, dtype=jnp.float32)


def _tc_body(x_ref, o_ref):
    nb = BLK2 // BLK
    blk_of = lax.broadcasted_iota(jnp.int32, (nb, BLK2), 1) // BLK
    sel = (blk_of == lax.broadcasted_iota(jnp.int32, (nb, BLK2), 0))
    o_ref[...] = jax.lax.dot(
        sel.astype(jnp.float32), x_ref[...],
        preferred_element_type=jnp.float32)


_tc_block_sums = pl.pallas_call(
    _tc_body,
    grid=(TC_ROWS // BLK2,),
    in_specs=[pl.BlockSpec((BLK2, D),
                           lambda i: (i + SC_ROWS // BLK2, i * 0))],
    out_specs=pl.BlockSpec((BLK2 // BLK, D), lambda i: (i, i * 0)),
    out_shape=jax.ShapeDtypeStruct((TC_NBLK, D), jnp.float32),
)


@functools.partial(
    pl.kernel,
    out_type=jax.ShapeDtypeStruct((SC_NBLK, D), jnp.float32),
    mesh=_mesh,
    scratch_types=[
        pltpu.VMEM((2, CHUNK, D), jnp.float32),
        pltpu.VMEM((BLK_PER_W, D), jnp.float32),
        pltpu.SemaphoreType.DMA,
        pltpu.SemaphoreType.DMA,
    ],
)
def _block_sums(x_hbm, bs_hbm, buf, acc_v, sem0, sem1):
    wid = lax.axis_index("s") * NC + lax.axis_index("c")
    row0 = wid * ROWS_PER_W
    sems = (sem0, sem1)
    copies = [None, None]
    NCH = ROWS_PER_W // CHUNK
    copies[0] = pltpu.async_copy(
        x_hbm.at[pl.ds(row0, CHUNK)], buf.at[_i32(0)], sem0)
    for g in range(NCH):
        cur = g % 2
        if g + 1 < NCH:
            nxt = (g + 1) % 2
            copies[nxt] = pltpu.async_copy(
                x_hbm.at[pl.ds(row0 + (g + 1) * CHUNK, CHUNK)],
                buf.at[_i32(nxt)], sems[nxt])
        copies[cur].wait()
        bb = buf.at[_i32(cur)]
        for sb in range(CHUNK // BLK):

            def body(r, accs, _sb=sb):
                r2 = r + r + _sb * BLK
                r3 = r2 + 1
                accs = tuple(accs[c] + bb[r2, pl.ds(c * L, L)]
                             for c in range(D // L))
                return tuple(accs[c] + bb[r3, pl.ds(c * L, L)]
                             for c in range(D // L))

            accs = lax.fori_loop(
                _i32(0), _i32(BLK // 2), body,
                tuple(jnp.zeros((L,), jnp.float32) for _ in range(D // L)))
            b = g * (CHUNK // BLK) + sb
            for c in range(D // L):
                acc_v[_i32(b), pl.ds(c * L, L)] = accs[c]
    pltpu.sync_copy(acc_v, bs_hbm.at[pl.ds(wid * BLK_PER_W, BLK_PER_W)])


@functools.partial(
    pl.kernel,
    out_type=jax.ShapeDtypeStruct((B, D), jnp.float32),
    mesh=_mesh,
    scratch_types=[
        pltpu.VMEM((9, L), jnp.int32),
        pltpu.VMEM((NBLK, DH), jnp.float32),
        pltpu.VMEM((EDGE, DH), jnp.float32),
        pltpu.VMEM((EDGE, DH), jnp.float32),
        pltpu.VMEM((DH,), jnp.float32),
        pltpu.VMEM_SHARED((B // NC, D), jnp.float32),
        pltpu.SemaphoreType.DMA,
        pltpu.SemaphoreType.DMA,
        pltpu.SemaphoreType.DMA,
    ],
    compiler_params=pltpu.CompilerParams(needs_layout_passes=False),
)
def _combine(x_hbm, bs_sc_hbm, bs_tc_hbm, pi_hbm, out_hbm,
             pv, bsv, e1v, e2v, outv, slab, sem_bs, sem_e1, sem_e2):
    sc = lax.axis_index("c")
    sid = lax.axis_index("s")
    lw = sc * NS + sid
    m = lw // 2
    h = lw % 2
    col0 = h * DH

    cp_bs1 = pltpu.async_copy(
        bs_sc_hbm.at[pl.ds(0, SC_NBLK), pl.ds(col0, DH)],
        bsv.at[pl.ds(_i32(0), SC_NBLK)], sem_bs)
    cp_bs2 = pltpu.async_copy(
        bs_tc_hbm.at[pl.ds(0, TC_NBLK), pl.ds(col0, DH)],
        bsv.at[pl.ds(_i32(SC_NBLK), TC_NBLK)], sem_bs)
    pltpu.sync_copy(pi_hbm, pv)
    fb_lo = _lane_i32(pv[_i32(0)], m)
    fb_hi = _lane_i32(pv[_i32(1)], m)
    e1_lo = _lane_i32(pv[_i32(2)], m)
    e1_hi = _lane_i32(pv[_i32(3)], m)
    e2_lo = _lane_i32(pv[_i32(4)], m)
    e2_hi = _lane_i32(pv[_i32(5)], m)
    c1 = pl.multiple_of(_lane_i32(pv[_i32(6)], m), 8)
    c2 = pl.multiple_of(_lane_i32(pv[_i32(7)], m), 8)
    inv = _lane_f32(plsc.bitcast(pv[_i32(8)], jnp.float32), m)

    cp_e1 = pltpu.async_copy(
        x_hbm.at[pl.ds(c1, EDGE), pl.ds(col0, DH)], e1v, sem_e1)
    cp_e2 = pltpu.async_copy(
        x_hbm.at[pl.ds(c2, EDGE), pl.ds(col0, DH)], e2v, sem_e2)

    zero = tuple(jnp.zeros((L,), jnp.float32) for _ in range(DH // L))

    cp_bs1.wait()
    cp_bs2.wait()

    def fb_body(bk, accs):
        return tuple(accs[c] + bsv[bk, pl.ds(c * L, L)]
                     for c in range(DH // L))

    accs = lax.fori_loop(fb_lo, fb_hi, fb_body, zero)

    cp_e1.wait()

    def e1_body(r, accs):
        return tuple(accs[c] + e1v[r, pl.ds(c * L, L)]
                     for c in range(DH // L))

    accs = lax.fori_loop(e1_lo - c1, e1_hi - c1, e1_body, accs)

    cp_e2.wait()

    def e2_body(r, accs):
        return tuple(accs[c] + e2v[r, pl.ds(c * L, L)]
                     for c in range(DH // L))

    accs = lax.fori_loop(e2_lo - c2, e2_hi - c2, e2_body, accs)

    for c in range(DH // L):
        outv[pl.ds(c * L, L)] = accs[c] * inv
    pltpu.sync_copy(outv, slab.at[m % (B // NC), pl.ds(col0, DH)])
    plsc.subcore_barrier()

    @pl.when(sid == 0)
    def _():
        pltpu.sync_copy(slab, out_hbm.at[pl.ds(sc * (B // NC), B // NC)])


def kernel(atom_hiddens, a_scope):
    x = atom_hiddens.astype(jnp.float32)
    s = a_scope[:, 0].astype(jnp.int32)
    sz = a_scope[:, 1].astype(jnp.int32)
    e = jnp.minimum(s + sz, N)
    b0 = (s + BLK - 1) // BLK
    b1 = e // BLK
    has_full = b0 < b1
    fb_lo = jnp.where(has_full, b0, 0)
    fb_hi = jnp.where(has_full, b1, 0)
    e1_lo = s
    e1_hi = jnp.where(has_full, b0 * BLK, e)
    e2_lo = jnp.where(has_full, b1 * BLK, 0)
    e2_hi = jnp.where(has_full, e, 0)
    c1 = jnp.minimum((e1_lo // 8) * 8, N - EDGE)
    c2 = jnp.minimum((e2_lo // 8) * 8, N - EDGE)
    inv = jnp.where(sz > 0, 1.0 / jnp.maximum(sz, 1).astype(jnp.float32), 0.0)
    inv_bits = lax.bitcast_convert_type(inv.astype(jnp.float32), jnp.int32)
    pi = jnp.stack([fb_lo, fb_hi, e1_lo, e1_hi, e2_lo, e2_hi, c1, c2,
                    inv_bits])

    bs_sc = _block_sums(x)
    bs_tc = _tc_block_sums(x)
    return _combine(x, bs_sc, bs_tc, pi)


# speedup vs baseline: 1.2273x; 1.0069x over previous
import functools

import jax
import jax.numpy as jnp
from jax import lax
from jax.experimental import pallas as pl
from jax.experimental.pallas import tpu as pltpu
from jax.experimental.pallas import tpu_sc as plsc

N = 32768
D = 256
B = 16
L = 16
NC, NS = 2, 16
NW = NC * NS
BLK = 64
NBLK = N // BLK
SC_ROWS = 16384
TC_ROWS = N - SC_ROWS
SC_NBLK = SC_ROWS // BLK
TC_NBLK = TC_ROWS // BLK
BLK_PER_W = SC_NBLK // NW
ROWS_PER_W = SC_ROWS // NW
BLK2 = 2048
DH = D // 2
EDGE = 2 * BLK + 8
CHUNK = 2 * BLK

_mesh = plsc.VectorSubcoreMesh(core_axis_name="c", subcore_axis_name="s")


def _i32(v):
    return jnp.asarray(v, jnp.int32)


def _lane_i32(vec, m):
    mask = (lax.iota(jnp.int32, L) == m).astype(jnp.int32)
    return jnp.sum(vec * mask, dtype=jnp.int32)


def _lane_f32(vec, m):
    mask = (lax.iota(jnp.int32, L) == m).astype(jnp.float32)
    return jnp.sum(vec * mask, dtype=jnp.float32)


def _tc_body(x_ref, o_ref):
    nb = BLK2 // BLK
    blk_of = lax.broadcasted_iota(jnp.int32, (nb, BLK2), 1) // BLK
    sel = (blk_of == lax.broadcasted_iota(jnp.int32, (nb, BLK2), 0))
    o_ref[...] = jax.lax.dot(
        sel.astype(jnp.float32), x_ref[...],
        preferred_element_type=jnp.float32)


_tc_block_sums = pl.pallas_call(
    _tc_body,
    grid=(TC_ROWS // BLK2,),
    in_specs=[pl.BlockSpec((BLK2, D),
                           lambda i: (i + SC_ROWS // BLK2, i * 0))],
    out_specs=pl.BlockSpec((BLK2 // BLK, D), lambda i: (i, i * 0)),
    out_shape=jax.ShapeDtypeStruct((TC_NBLK, D), jnp.float32),
)


@functools.partial(
    pl.kernel,
    out_type=jax.ShapeDtypeStruct((SC_NBLK, D), jnp.float32),
    mesh=_mesh,
    scratch_types=[
        pltpu.VMEM((2, CHUNK, D), jnp.float32),
        pltpu.VMEM((BLK_PER_W, D), jnp.float32),
        pltpu.SemaphoreType.DMA,
        pltpu.SemaphoreType.DMA,
    ],
)
def _block_sums(x_hbm, bs_hbm, buf, acc_v, sem0, sem1):
    wid = lax.axis_index("s") * NC + lax.axis_index("c")
    row0 = wid * ROWS_PER_W
    sems = (sem0, sem1)
    copies = [None, None]
    NCH = ROWS_PER_W // CHUNK
    copies[0] = pltpu.async_copy(
        x_hbm.at[pl.ds(row0, CHUNK)], buf.at[_i32(0)], sem0)
    for g in range(NCH):
        cur = g % 2
        if g + 1 < NCH:
            nxt = (g + 1) % 2
            copies[nxt] = pltpu.async_copy(
                x_hbm.at[pl.ds(row0 + (g + 1) * CHUNK, CHUNK)],
                buf.at[_i32(nxt)], sems[nxt])
        copies[cur].wait()
        bb = buf.at[_i32(cur)]
        for sb in range(CHUNK // BLK):

            def body(r, accs, _sb=sb):
                r2 = r + r + _sb * BLK
                r3 = r2 + 1
                accs = tuple(accs[c] + bb[r2, pl.ds(c * L, L)]
                             for c in range(D // L))
                return tuple(accs[c] + bb[r3, pl.ds(c * L, L)]
                             for c in range(D // L))

            accs = lax.fori_loop(
                _i32(0), _i32(BLK // 2), body,
                tuple(jnp.zeros((L,), jnp.float32) for _ in range(D // L)))
            b = g * (CHUNK // BLK) + sb
            for c in range(D // L):
                acc_v[_i32(b), pl.ds(c * L, L)] = accs[c]
    pltpu.sync_copy(acc_v, bs_hbm.at[pl.ds(wid * BLK_PER_W, BLK_PER_W)])


@functools.partial(
    pl.kernel,
    out_type=jax.ShapeDtypeStruct((B, D), jnp.float32),
    mesh=_mesh,
    scratch_types=[
        pltpu.VMEM((2, L), jnp.int32),
        pltpu.VMEM((NBLK, DH), jnp.float32),
        pltpu.VMEM((EDGE, DH), jnp.float32),
        pltpu.VMEM((EDGE, DH), jnp.float32),
        pltpu.VMEM((DH,), jnp.float32),
        pltpu.VMEM_SHARED((B // NC, D), jnp.float32),
        pltpu.SemaphoreType.DMA,
        pltpu.SemaphoreType.DMA,
        pltpu.SemaphoreType.DMA,
    ],
    compiler_params=pltpu.CompilerParams(needs_layout_passes=False),
)
def _combine(x_hbm, bs_sc_hbm, bs_tc_hbm, scope_hbm, out_hbm,
             pv, bsv, e1v, e2v, outv, slab, sem_bs, sem_e1, sem_e2):
    sc = lax.axis_index("c")
    sid = lax.axis_index("s")
    lw = sc * NS + sid
    m = lw // 2
    h = lw % 2
    col0 = h * DH

    cp_bs1 = pltpu.async_copy(
        bs_sc_hbm.at[pl.ds(0, SC_NBLK), pl.ds(col0, DH)],
        bsv.at[pl.ds(_i32(0), SC_NBLK)], sem_bs)
    cp_bs2 = pltpu.async_copy(
        bs_tc_hbm.at[pl.ds(0, TC_NBLK), pl.ds(col0, DH)],
        bsv.at[pl.ds(_i32(SC_NBLK), TC_NBLK)], sem_bs)
    pltpu.sync_copy(scope_hbm, pv)
    sv = pv[_i32(0)]
    zv = pv[_i32(1)]
    ev = jnp.minimum(sv + zv, N)
    b0v = lax.shift_right_logical(sv + (BLK - 1), _i32(6))
    b1v = lax.shift_right_logical(ev, _i32(6))
    hf = b0v < b1v
    zero_v = jnp.zeros((L,), jnp.int32)
    fb_lo_v = jnp.where(hf, b0v, zero_v)
    fb_hi_v = jnp.where(hf, b1v, zero_v)
    e1_hi_v = jnp.where(hf, lax.shift_left(b0v, _i32(6)), ev)
    e2_lo_v = jnp.where(hf, lax.shift_left(b1v, _i32(6)), zero_v)
    e2_hi_v = jnp.where(hf, ev, zero_v)
    c1v = jnp.minimum(
        lax.shift_left(lax.shift_right_logical(sv, _i32(3)), _i32(3)), N - EDGE)
    c2v = jnp.minimum(
        lax.shift_left(lax.shift_right_logical(e2_lo_v, _i32(3)), _i32(3)), N - EDGE)
    szf = zv.astype(jnp.float32)
    inv_v = jnp.where(zv > 0, 1.0 / jnp.maximum(szf, 1.0),
                      jnp.zeros((L,), jnp.float32))
    fb_lo = _lane_i32(fb_lo_v, m)
    fb_hi = _lane_i32(fb_hi_v, m)
    e1_lo = _lane_i32(sv, m)
    e1_hi = _lane_i32(e1_hi_v, m)
    e2_lo = _lane_i32(e2_lo_v, m)
    e2_hi = _lane_i32(e2_hi_v, m)
    c1 = pl.multiple_of(_lane_i32(c1v, m), 8)
    c2 = pl.multiple_of(_lane_i32(c2v, m), 8)
    inv = _lane_f32(inv_v, m)

    cp_e1 = pltpu.async_copy(
        x_hbm.at[pl.ds(c1, EDGE), pl.ds(col0, DH)], e1v, sem_e1)
    cp_e2 = pltpu.async_copy(
        x_hbm.at[pl.ds(c2, EDGE), pl.ds(col0, DH)], e2v, sem_e2)

    zero = tuple(jnp.zeros((L,), jnp.float32) for _ in range(DH // L))

    cp_bs1.wait()
    cp_bs2.wait()

    def fb_body(bk, accs):
        return tuple(accs[c] + bsv[bk, pl.ds(c * L, L)]
                     for c in range(DH // L))

    accs = lax.fori_loop(fb_lo, fb_hi, fb_body, zero)

    cp_e1.wait()

    def e1_body(r, accs):
        return tuple(accs[c] + e1v[r, pl.ds(c * L, L)]
                     for c in range(DH // L))

    accs = lax.fori_loop(e1_lo - c1, e1_hi - c1, e1_body, accs)

    cp_e2.wait()

    def e2_body(r, accs):
        return tuple(accs[c] + e2v[r, pl.ds(c * L, L)]
                     for c in range(DH // L))

    accs = lax.fori_loop(e2_lo - c2, e2_hi - c2, e2_body, accs)

    for c in range(DH // L):
        outv[pl.ds(c * L, L)] = accs[c] * inv
    pltpu.sync_copy(outv, slab.at[m % (B // NC), pl.ds(col0, DH)])
    plsc.subcore_barrier()

    @pl.when(sid == 0)
    def _():
        pltpu.sync_copy(slab, out_hbm.at[pl.ds(sc * (B // NC), B // NC)])


def kernel(atom_hiddens, a_scope):
    x = atom_hiddens.astype(jnp.float32)
    scope2 = a_scope.astype(jnp.int32).T
    bs_sc = _block_sums(x)
    bs_tc = _tc_block_sums(x)
    return _combine(x, bs_sc, bs_tc, scope2)


# speedup vs baseline: 1.2375x; 1.0083x over previous
import functools

import jax
import jax.numpy as jnp
from jax import lax
from jax.experimental import pallas as pl
from jax.experimental.pallas import tpu as pltpu
from jax.experimental.pallas import tpu_sc as plsc

N = 32768
D = 256
B = 16
L = 16
NC, NS = 2, 16
NW = NC * NS
BLK = 64
NBLK = N // BLK
SC_ROWS = 16384
TC_ROWS = N - SC_ROWS
SC_NBLK = SC_ROWS // BLK
TC_NBLK = TC_ROWS // BLK
BLK_PER_W = SC_NBLK // NW
ROWS_PER_W = SC_ROWS // NW
BLK2 = 2048
DH = D // 2
EDGE = 2 * BLK + 8
CHUNK = 2 * BLK

_mesh = plsc.VectorSubcoreMesh(core_axis_name="c", subcore_axis_name="s")


def _i32(v):
    return jnp.asarray(v, jnp.int32)


def _lane_i32(vec, m):
    mask = (lax.iota(jnp.int32, L) == m).astype(jnp.int32)
    return jnp.sum(vec * mask, dtype=jnp.int32)


def _lane_f32(vec, m):
    mask = (lax.iota(jnp.int32, L) == m).astype(jnp.float32)
    return jnp.sum(vec * mask, dtype=jnp.float32)


def _tc_body(x_ref, o_ref):
    nb = BLK2 // BLK
    blk_of = lax.broadcasted_iota(jnp.int32, (nb, BLK2), 1) // BLK
    sel = (blk_of == lax.broadcasted_iota(jnp.int32, (nb, BLK2), 0))
    o_ref[...] = jax.lax.dot(
        sel.astype(jnp.float32), x_ref[...],
        preferred_element_type=jnp.float32)


_tc_block_sums = pl.pallas_call(
    _tc_body,
    grid=(TC_ROWS // BLK2,),
    in_specs=[pl.BlockSpec((BLK2, D),
                           lambda i: (i + SC_ROWS // BLK2, i * 0))],
    out_specs=pl.BlockSpec((BLK2 // BLK, D), lambda i: (i, i * 0)),
    out_shape=jax.ShapeDtypeStruct((TC_NBLK, D), jnp.float32),
)


@functools.partial(
    pl.kernel,
    out_type=jax.ShapeDtypeStruct((SC_NBLK, D), jnp.float32),
    mesh=_mesh,
    scratch_types=[
        pltpu.VMEM((2, CHUNK, D), jnp.float32),
        pltpu.VMEM((BLK_PER_W, D), jnp.float32),
        pltpu.SemaphoreType.DMA,
        pltpu.SemaphoreType.DMA,
    ],
)
def _block_sums(x_hbm, bs_hbm, buf, acc_v, sem0, sem1):
    wid = lax.axis_index("s") * NC + lax.axis_index("c")
    row0 = wid * ROWS_PER_W
    sems = (sem0, sem1)
    copies = [None, None]
    NCH = ROWS_PER_W // CHUNK
    copies[0] = pltpu.async_copy(
        x_hbm.at[pl.ds(row0, CHUNK)], buf.at[_i32(0)], sem0)
    for g in range(NCH):
        cur = g % 2
        if g + 1 < NCH:
            nxt = (g + 1) % 2
            copies[nxt] = pltpu.async_copy(
                x_hbm.at[pl.ds(row0 + (g + 1) * CHUNK, CHUNK)],
                buf.at[_i32(nxt)], sems[nxt])
        copies[cur].wait()
        bb = buf.at[_i32(cur)]
        for sb in range(CHUNK // BLK):

            def body(r, accs, _sb=sb):
                r2 = r + r + _sb * BLK
                r3 = r2 + 1
                accs = tuple(accs[c] + bb[r2, pl.ds(c * L, L)]
                             for c in range(D // L))
                return tuple(accs[c] + bb[r3, pl.ds(c * L, L)]
                             for c in range(D // L))

            accs = lax.fori_loop(
                _i32(0), _i32(BLK // 2), body,
                tuple(jnp.zeros((L,), jnp.float32) for _ in range(D // L)))
            b = g * (CHUNK // BLK) + sb
            for c in range(D // L):
                acc_v[_i32(b), pl.ds(c * L, L)] = accs[c]
    pltpu.sync_copy(acc_v, bs_hbm.at[pl.ds(wid * BLK_PER_W, BLK_PER_W)])


@functools.partial(
    pl.kernel,
    out_type=jax.ShapeDtypeStruct((B, D), jnp.float32),
    mesh=_mesh,
    scratch_types=[
        pltpu.VMEM((2, L), jnp.int32),
        pltpu.VMEM((NBLK, DH), jnp.float32),
        pltpu.VMEM((EDGE, DH), jnp.float32),
        pltpu.VMEM((EDGE, DH), jnp.float32),
        pltpu.VMEM((DH,), jnp.float32),
        pltpu.VMEM_SHARED((B // NC, D), jnp.float32),
        pltpu.SemaphoreType.DMA,
        pltpu.SemaphoreType.DMA,
        pltpu.SemaphoreType.DMA,
        pltpu.SemaphoreType.DMA,
    ],
    compiler_params=pltpu.CompilerParams(needs_layout_passes=False),
)
def _combine(x_hbm, bs_sc_hbm, bs_tc_hbm, scope_hbm, out_hbm,
             pv, bsv, e1v, e2v, outv, slab, sem_bs, sem_bs2, sem_e1, sem_e2):
    sc = lax.axis_index("c")
    sid = lax.axis_index("s")
    lw = sc * NS + sid
    m = lw // 2
    h = lw % 2
    col0 = h * DH

    cp_bs1 = pltpu.async_copy(
        bs_sc_hbm.at[pl.ds(0, SC_NBLK), pl.ds(col0, DH)],
        bsv.at[pl.ds(_i32(0), SC_NBLK)], sem_bs)
    cp_bs2 = pltpu.async_copy(
        bs_tc_hbm.at[pl.ds(0, TC_NBLK), pl.ds(col0, DH)],
        bsv.at[pl.ds(_i32(SC_NBLK), TC_NBLK)], sem_bs2)
    pltpu.sync_copy(scope_hbm, pv)
    sv = pv[_i32(0)]
    zv = pv[_i32(1)]
    ev = jnp.minimum(sv + zv, N)
    b0v = lax.shift_right_logical(sv + (BLK - 1), _i32(6))
    b1v = lax.shift_right_logical(ev, _i32(6))
    hf = b0v < b1v
    zero_v = jnp.zeros((L,), jnp.int32)
    fb_lo_v = jnp.where(hf, b0v, zero_v)
    fb_hi_v = jnp.where(hf, b1v, zero_v)
    e1_hi_v = jnp.where(hf, lax.shift_left(b0v, _i32(6)), ev)
    e2_lo_v = jnp.where(hf, lax.shift_left(b1v, _i32(6)), zero_v)
    e2_hi_v = jnp.where(hf, ev, zero_v)
    c1v = jnp.minimum(
        lax.shift_left(lax.shift_right_logical(sv, _i32(3)), _i32(3)), N - EDGE)
    c2v = jnp.minimum(
        lax.shift_left(lax.shift_right_logical(e2_lo_v, _i32(3)), _i32(3)), N - EDGE)
    szf = zv.astype(jnp.float32)
    inv_v = jnp.where(zv > 0, 1.0 / jnp.maximum(szf, 1.0),
                      jnp.zeros((L,), jnp.float32))
    fb_lo = _lane_i32(fb_lo_v, m)
    fb_hi = _lane_i32(fb_hi_v, m)
    e1_lo = _lane_i32(sv, m)
    e1_hi = _lane_i32(e1_hi_v, m)
    e2_lo = _lane_i32(e2_lo_v, m)
    e2_hi = _lane_i32(e2_hi_v, m)
    c1 = pl.multiple_of(_lane_i32(c1v, m), 8)
    c2 = pl.multiple_of(_lane_i32(c2v, m), 8)
    inv = _lane_f32(inv_v, m)

    cp_e1 = pltpu.async_copy(
        x_hbm.at[pl.ds(c1, EDGE), pl.ds(col0, DH)], e1v, sem_e1)
    cp_e2 = pltpu.async_copy(
        x_hbm.at[pl.ds(c2, EDGE), pl.ds(col0, DH)], e2v, sem_e2)

    zero = tuple(jnp.zeros((L,), jnp.float32) for _ in range(DH // L))

    def fb_body(bk, accs):
        return tuple(accs[c] + bsv[bk, pl.ds(c * L, L)]
                     for c in range(DH // L))

    cp_bs1.wait()
    accs = lax.fori_loop(jnp.minimum(fb_lo, SC_NBLK),
                         jnp.minimum(fb_hi, SC_NBLK), fb_body, zero)
    cp_bs2.wait()
    accs = lax.fori_loop(jnp.maximum(fb_lo, SC_NBLK),
                         jnp.maximum(fb_hi, SC_NBLK), fb_body, accs)

    cp_e1.wait()

    def e1_body(r, accs):
        return tuple(accs[c] + e1v[r, pl.ds(c * L, L)]
                     for c in range(DH // L))

    accs = lax.fori_loop(e1_lo - c1, e1_hi - c1, e1_body, accs)

    cp_e2.wait()

    def e2_body(r, accs):
        return tuple(accs[c] + e2v[r, pl.ds(c * L, L)]
                     for c in range(DH // L))

    accs = lax.fori_loop(e2_lo - c2, e2_hi - c2, e2_body, accs)

    for c in range(DH // L):
        outv[pl.ds(c * L, L)] = accs[c] * inv
    pltpu.sync_copy(outv, slab.at[m % (B // NC), pl.ds(col0, DH)])
    plsc.subcore_barrier()

    @pl.when(sid == 0)
    def _():
        pltpu.sync_copy(slab, out_hbm.at[pl.ds(sc * (B // NC), B // NC)])


def kernel(atom_hiddens, a_scope):
    x = atom_hiddens.astype(jnp.float32)
    scope2 = a_scope.astype(jnp.int32).T
    bs_sc = _block_sums(x)
    bs_tc = _tc_block_sums(x)
    return _combine(x, bs_sc, bs_tc, scope2)


# speedup vs baseline: 1.2715x; 1.0275x over previous
import functools

import jax
import jax.numpy as jnp
from jax import lax
from jax.experimental import pallas as pl
from jax.experimental.pallas import tpu as pltpu
from jax.experimental.pallas import tpu_sc as plsc

N = 32768
D = 256
B = 16
L = 16
NC, NS = 2, 16
NW = NC * NS
BLK = 64
NBLK = N // BLK
SC_ROWS = 16384
TC_ROWS = N - SC_ROWS
SC_NBLK = SC_ROWS // BLK
TC_NBLK = TC_ROWS // BLK
BLK_PER_W = SC_NBLK // NW
ROWS_PER_W = SC_ROWS // NW
BLK2 = 2048
DH = D // 2
EDGE = 2 * BLK + 8
CHUNK = 2 * BLK

_mesh = plsc.VectorSubcoreMesh(core_axis_name="c", subcore_axis_name="s")


def _i32(v):
    return jnp.asarray(v, jnp.int32)


def _lane_i32(vec, m):
    mask = (lax.iota(jnp.int32, L) == m).astype(jnp.int32)
    return jnp.sum(vec * mask, dtype=jnp.int32)


def _lane_f32(vec, m):
    mask = (lax.iota(jnp.int32, L) == m).astype(jnp.float32)
    return jnp.sum(vec * mask, dtype=jnp.float32)


def _tc_body(s_ref, e_ref, x_ref, o_ref):
    g0 = pl.program_id(0) * BLK2 + SC_ROWS
    row = lax.broadcasted_iota(jnp.int32, (B, BLK2), 1) + g0
    sel = ((row >= s_ref[...]) & (row < e_ref[...])).astype(jnp.float32)
    part = jax.lax.dot(sel, x_ref[...], preferred_element_type=jnp.float32)

    @pl.when(pl.program_id(0) == 0)
    def _():
        o_ref[...] = part

    @pl.when(pl.program_id(0) > 0)
    def _():
        o_ref[...] += part


_tc_partials = pl.pallas_call(
    _tc_body,
    grid=(TC_ROWS // BLK2,),
    in_specs=[
        pl.BlockSpec((B, 1), lambda i: (i * 0, i * 0)),
        pl.BlockSpec((B, 1), lambda i: (i * 0, i * 0)),
        pl.BlockSpec((BLK2, D), lambda i: (i + SC_ROWS // BLK2, i * 0)),
    ],
    out_specs=pl.BlockSpec((B, D), lambda i: (i * 0, i * 0)),
    out_shape=jax.ShapeDtypeStruct((B, D), jnp.float32),
)


@functools.partial(
    pl.kernel,
    out_type=jax.ShapeDtypeStruct((SC_NBLK, D), jnp.float32),
    mesh=_mesh,
    scratch_types=[
        pltpu.VMEM((2, CHUNK, D), jnp.float32),
        pltpu.VMEM((BLK_PER_W, D), jnp.float32),
        pltpu.SemaphoreType.DMA,
        pltpu.SemaphoreType.DMA,
    ],
)
def _block_sums(x_hbm, bs_hbm, buf, acc_v, sem0, sem1):
    wid = lax.axis_index("s") * NC + lax.axis_index("c")
    row0 = wid * ROWS_PER_W
    sems = (sem0, sem1)
    copies = [None, None]
    NCH = ROWS_PER_W // CHUNK
    copies[0] = pltpu.async_copy(
        x_hbm.at[pl.ds(row0, CHUNK)], buf.at[_i32(0)], sem0)
    for g in range(NCH):
        cur = g % 2
        if g + 1 < NCH:
            nxt = (g + 1) % 2
            copies[nxt] = pltpu.async_copy(
                x_hbm.at[pl.ds(row0 + (g + 1) * CHUNK, CHUNK)],
                buf.at[_i32(nxt)], sems[nxt])
        copies[cur].wait()
        bb = buf.at[_i32(cur)]
        for sb in range(CHUNK // BLK):

            def body(r, accs, _sb=sb):
                r2 = r + r + _sb * BLK
                r3 = r2 + 1
                accs = tuple(accs[c] + bb[r2, pl.ds(c * L, L)]
                             for c in range(D // L))
                return tuple(accs[c] + bb[r3, pl.ds(c * L, L)]
                             for c in range(D // L))

            accs = lax.fori_loop(
                _i32(0), _i32(BLK // 2), body,
                tuple(jnp.zeros((L,), jnp.float32) for _ in range(D // L)))
            b = g * (CHUNK // BLK) + sb
            for c in range(D // L):
                acc_v[_i32(b), pl.ds(c * L, L)] = accs[c]
    pltpu.sync_copy(acc_v, bs_hbm.at[pl.ds(wid * BLK_PER_W, BLK_PER_W)])


@functools.partial(
    pl.kernel,
    out_type=jax.ShapeDtypeStruct((B, D), jnp.float32),
    mesh=_mesh,
    scratch_types=[
        pltpu.VMEM((2, L), jnp.int32),
        pltpu.VMEM((SC_NBLK, DH), jnp.float32),
        pltpu.VMEM((B, D), jnp.float32),
        pltpu.VMEM((EDGE, DH), jnp.float32),
        pltpu.VMEM((EDGE, DH), jnp.float32),
        pltpu.VMEM((DH,), jnp.float32),
        pltpu.VMEM_SHARED((B // NC, D), jnp.float32),
        pltpu.SemaphoreType.DMA,
        pltpu.SemaphoreType.DMA,
        pltpu.SemaphoreType.DMA,
        pltpu.SemaphoreType.DMA,
    ],
    compiler_params=pltpu.CompilerParams(needs_layout_passes=False),
)
def _combine(x_hbm, bs_sc_hbm, tcp_hbm, scope_hbm, out_hbm,
             pv, bsv, tpv, e1v, e2v, outv, slab,
             sem_bs, sem_bs2, sem_e1, sem_e2):
    sc = lax.axis_index("c")
    sid = lax.axis_index("s")
    lw = sc * NS + sid
    m = lw // 2
    h = lw % 2
    col0 = h * DH

    cp_bs1 = pltpu.async_copy(
        bs_sc_hbm.at[pl.ds(0, SC_NBLK), pl.ds(col0, DH)], bsv, sem_bs)
    cp_tp = pltpu.async_copy(tcp_hbm, tpv, sem_bs2)
    pltpu.sync_copy(scope_hbm, pv)
    sv = pv[_i32(0)]
    zv = pv[_i32(1)]
    ev = jnp.minimum(sv + zv, SC_ROWS)
    b0v = lax.shift_right_logical(sv + (BLK - 1), _i32(6))
    b1v = lax.shift_right_logical(ev, _i32(6))
    hf = b0v < b1v
    zero_v = jnp.zeros((L,), jnp.int32)
    fb_lo_v = jnp.where(hf, b0v, zero_v)
    fb_hi_v = jnp.where(hf, b1v, zero_v)
    e1_hi_v = jnp.where(hf, lax.shift_left(b0v, _i32(6)), ev)
    e2_lo_v = jnp.where(hf, lax.shift_left(b1v, _i32(6)), zero_v)
    e2_hi_v = jnp.where(hf, ev, zero_v)
    c1v = jnp.minimum(
        lax.shift_left(lax.shift_right_logical(sv, _i32(3)), _i32(3)), N - EDGE)
    c2v = jnp.minimum(
        lax.shift_left(lax.shift_right_logical(e2_lo_v, _i32(3)), _i32(3)), N - EDGE)
    szf = zv.astype(jnp.float32)
    inv_v = jnp.where(zv > 0, 1.0 / jnp.maximum(szf, 1.0),
                      jnp.zeros((L,), jnp.float32))
    fb_lo = _lane_i32(fb_lo_v, m)
    fb_hi = _lane_i32(fb_hi_v, m)
    e1_lo = _lane_i32(sv, m)
    e1_hi = _lane_i32(e1_hi_v, m)
    e2_lo = _lane_i32(e2_lo_v, m)
    e2_hi = _lane_i32(e2_hi_v, m)
    c1 = pl.multiple_of(_lane_i32(c1v, m), 8)
    c2 = pl.multiple_of(_lane_i32(c2v, m), 8)
    inv = _lane_f32(inv_v, m)

    cp_e1 = pltpu.async_copy(
        x_hbm.at[pl.ds(c1, EDGE), pl.ds(col0, DH)], e1v, sem_e1)
    cp_e2 = pltpu.async_copy(
        x_hbm.at[pl.ds(c2, EDGE), pl.ds(col0, DH)], e2v, sem_e2)

    zero = tuple(jnp.zeros((L,), jnp.float32) for _ in range(DH // L))

    def fb_body(bk, accs):
        return tuple(accs[c] + bsv[bk, pl.ds(c * L, L)]
                     for c in range(DH // L))

    cp_bs1.wait()
    accs = lax.fori_loop(fb_lo, fb_hi, fb_body, zero)

    cp_e1.wait()

    def e1_body(r, accs):
        return tuple(accs[c] + e1v[r, pl.ds(c * L, L)]
                     for c in range(DH // L))

    accs = lax.fori_loop(e1_lo - c1, e1_hi - c1, e1_body, accs)

    cp_e2.wait()

    def e2_body(r, accs):
        return tuple(accs[c] + e2v[r, pl.ds(c * L, L)]
                     for c in range(DH // L))

    accs = lax.fori_loop(e2_lo - c2, e2_hi - c2, e2_body, accs)

    cp_tp.wait()
    accs = tuple(accs[c] + tpv[m, pl.ds(col0 + c * L, L)]
                 for c in range(DH // L))

    for c in range(DH // L):
        outv[pl.ds(c * L, L)] = accs[c] * inv
    pltpu.sync_copy(outv, slab.at[m % (B // NC), pl.ds(col0, DH)])
    plsc.subcore_barrier()

    @pl.when(sid == 0)
    def _():
        pltpu.sync_copy(slab, out_hbm.at[pl.ds(sc * (B // NC), B // NC)])


def kernel(atom_hiddens, a_scope):
    x = atom_hiddens.astype(jnp.float32)
    s32 = a_scope[:, 0].astype(jnp.int32)
    z32 = a_scope[:, 1].astype(jnp.int32)
    scope2 = jnp.stack([s32, z32])
    s_col = s32[:, None]
    e_col = jnp.minimum(s32 + z32, N)[:, None]
    bs_sc = _block_sums(x)
    tc_part = _tc_partials(s_col, e_col, x)
    return _combine(x, bs_sc, tc_part, scope2)


# speedup vs baseline: 1.2980x; 1.0208x over previous
import functools

import jax
import jax.numpy as jnp
from jax import lax
from jax.experimental import pallas as pl
from jax.experimental.pallas import tpu as pltpu
from jax.experimental.pallas import tpu_sc as plsc

N = 32768
D = 256
B = 16
L = 16
NC, NS = 2, 16
NW = NC * NS
BLK = 64
NBLK = N // BLK
SC_ROWS = 16384
TC_ROWS = N - SC_ROWS
SC_NBLK = SC_ROWS // BLK
TC_NBLK = TC_ROWS // BLK
BLK_PER_W = SC_NBLK // NW
ROWS_PER_W = SC_ROWS // NW
BLK2 = 2048
DH = D // 2
EDGE = 2 * BLK + 8
CHUNK = BLK
NBUF = 4

_mesh = plsc.VectorSubcoreMesh(core_axis_name="c", subcore_axis_name="s")


def _i32(v):
    return jnp.asarray(v, jnp.int32)


def _lane_i32(vec, m):
    mask = (lax.iota(jnp.int32, L) == m).astype(jnp.int32)
    return jnp.sum(vec * mask, dtype=jnp.int32)


def _lane_f32(vec, m):
    mask = (lax.iota(jnp.int32, L) == m).astype(jnp.float32)
    return jnp.sum(vec * mask, dtype=jnp.float32)


def _tc_body(s_ref, e_ref, x_ref, o_ref):
    g0 = pl.program_id(0) * BLK2 + SC_ROWS
    row = lax.broadcasted_iota(jnp.int32, (B, BLK2), 1) + g0
    sel = ((row >= s_ref[...]) & (row < e_ref[...])).astype(jnp.float32)
    part = jax.lax.dot(sel, x_ref[...], preferred_element_type=jnp.float32)

    @pl.when(pl.program_id(0) == 0)
    def _():
        o_ref[...] = part

    @pl.when(pl.program_id(0) > 0)
    def _():
        o_ref[...] += part


_tc_partials = pl.pallas_call(
    _tc_body,
    grid=(TC_ROWS // BLK2,),
    in_specs=[
        pl.BlockSpec((B, 1), lambda i: (i * 0, i * 0)),
        pl.BlockSpec((B, 1), lambda i: (i * 0, i * 0)),
        pl.BlockSpec((BLK2, D), lambda i: (i + SC_ROWS // BLK2, i * 0)),
    ],
    out_specs=pl.BlockSpec((B, D), lambda i: (i * 0, i * 0)),
    out_shape=jax.ShapeDtypeStruct((B, D), jnp.float32),
)


@functools.partial(
    pl.kernel,
    out_type=jax.ShapeDtypeStruct((SC_NBLK, D), jnp.float32),
    mesh=_mesh,
    scratch_types=[
        pltpu.VMEM((NBUF, CHUNK, D), jnp.float32),
        pltpu.VMEM((BLK_PER_W, D), jnp.float32),
        pltpu.SemaphoreType.DMA,
        pltpu.SemaphoreType.DMA,
        pltpu.SemaphoreType.DMA,
        pltpu.SemaphoreType.DMA,
    ],
)
def _block_sums(x_hbm, bs_hbm, buf, acc_v, sem0, sem1, sem2, sem3):
    wid = lax.axis_index("s") * NC + lax.axis_index("c")
    row0 = wid * ROWS_PER_W
    sems = (sem0, sem1, sem2, sem3)
    NCH = ROWS_PER_W // CHUNK
    copies = [None] * NBUF
    for k in range(NBUF - 1):
        copies[k] = pltpu.async_copy(
            x_hbm.at[pl.ds(row0 + k * CHUNK, CHUNK)], buf.at[_i32(k)],
            sems[k])
    for g in range(NCH):
        cur = g % NBUF
        nxt_g = g + NBUF - 1
        if nxt_g < NCH:
            nxt = nxt_g % NBUF
            copies[nxt] = pltpu.async_copy(
                x_hbm.at[pl.ds(row0 + nxt_g * CHUNK, CHUNK)],
                buf.at[_i32(nxt)], sems[nxt])
        copies[cur].wait()
        bb = buf.at[_i32(cur)]

        def body(r, accs):
            r2 = r + r
            r3 = r2 + 1
            accs = tuple(accs[c] + bb[r2, pl.ds(c * L, L)]
                         for c in range(D // L))
            return tuple(accs[c] + bb[r3, pl.ds(c * L, L)]
                         for c in range(D // L))

        accs = lax.fori_loop(
            _i32(0), _i32(BLK // 2), body,
            tuple(jnp.zeros((L,), jnp.float32) for _ in range(D // L)))
        for c in range(D // L):
            acc_v[_i32(g), pl.ds(c * L, L)] = accs[c]
    pltpu.sync_copy(acc_v, bs_hbm.at[pl.ds(wid * BLK_PER_W, BLK_PER_W)])


@functools.partial(
    pl.kernel,
    out_type=jax.ShapeDtypeStruct((B, D), jnp.float32),
    mesh=_mesh,
    scratch_types=[
        pltpu.VMEM((2, L), jnp.int32),
        pltpu.VMEM((SC_NBLK, DH), jnp.float32),
        pltpu.VMEM((B, D), jnp.float32),
        pltpu.VMEM((EDGE, DH), jnp.float32),
        pltpu.VMEM((EDGE, DH), jnp.float32),
        pltpu.VMEM((DH,), jnp.float32),
        pltpu.VMEM_SHARED((B // NC, D), jnp.float32),
        pltpu.SemaphoreType.DMA,
        pltpu.SemaphoreType.DMA,
        pltpu.SemaphoreType.DMA,
        pltpu.SemaphoreType.DMA,
    ],
    compiler_params=pltpu.CompilerParams(needs_layout_passes=False),
)
def _combine(x_hbm, bs_sc_hbm, tcp_hbm, scope_hbm, out_hbm,
             pv, bsv, tpv, e1v, e2v, outv, slab,
             sem_bs, sem_bs2, sem_e1, sem_e2):
    sc = lax.axis_index("c")
    sid = lax.axis_index("s")
    lw = sc * NS + sid
    m = lw // 2
    h = lw % 2
    col0 = h * DH

    cp_bs1 = pltpu.async_copy(
        bs_sc_hbm.at[pl.ds(0, SC_NBLK), pl.ds(col0, DH)], bsv, sem_bs)
    cp_tp = pltpu.async_copy(tcp_hbm, tpv, sem_bs2)
    pltpu.sync_copy(scope_hbm, pv)
    sv = pv[_i32(0)]
    zv = pv[_i32(1)]
    ev = jnp.minimum(sv + zv, SC_ROWS)
    b0v = lax.shift_right_logical(sv + (BLK - 1), _i32(6))
    b1v = lax.shift_right_logical(ev, _i32(6))
    hf = b0v < b1v
    zero_v = jnp.zeros((L,), jnp.int32)
    fb_lo_v = jnp.where(hf, b0v, zero_v)
    fb_hi_v = jnp.where(hf, b1v, zero_v)
    e1_hi_v = jnp.where(hf, lax.shift_left(b0v, _i32(6)), ev)
    e2_lo_v = jnp.where(hf, lax.shift_left(b1v, _i32(6)), zero_v)
    e2_hi_v = jnp.where(hf, ev, zero_v)
    c1v = jnp.minimum(
        lax.shift_left(lax.shift_right_logical(sv, _i32(3)), _i32(3)), N - EDGE)
    c2v = jnp.minimum(
        lax.shift_left(lax.shift_right_logical(e2_lo_v, _i32(3)), _i32(3)), N - EDGE)
    szf = zv.astype(jnp.float32)
    inv_v = jnp.where(zv > 0, 1.0 / jnp.maximum(szf, 1.0),
                      jnp.zeros((L,), jnp.float32))
    fb_lo = _lane_i32(fb_lo_v, m)
    fb_hi = _lane_i32(fb_hi_v, m)
    e1_lo = _lane_i32(sv, m)
    e1_hi = _lane_i32(e1_hi_v, m)
    e2_lo = _lane_i32(e2_lo_v, m)
    e2_hi = _lane_i32(e2_hi_v, m)
    c1 = pl.multiple_of(_lane_i32(c1v, m), 8)
    c2 = pl.multiple_of(_lane_i32(c2v, m), 8)
    inv = _lane_f32(inv_v, m)

    cp_e1 = pltpu.async_copy(
        x_hbm.at[pl.ds(c1, EDGE), pl.ds(col0, DH)], e1v, sem_e1)
    cp_e2 = pltpu.async_copy(
        x_hbm.at[pl.ds(c2, EDGE), pl.ds(col0, DH)], e2v, sem_e2)

    zero = tuple(jnp.zeros((L,), jnp.float32) for _ in range(DH // L))

    def fb_body(bk, accs):
        return tuple(accs[c] + bsv[bk, pl.ds(c * L, L)]
                     for c in range(DH // L))

    cp_bs1.wait()
    accs = lax.fori_loop(fb_lo, fb_hi, fb_body, zero)

    cp_e1.wait()

    def e1_body(r, accs):
        return tuple(accs[c] + e1v[r, pl.ds(c * L, L)]
                     for c in range(DH // L))

    accs = lax.fori_loop(e1_lo - c1, e1_hi - c1, e1_body, accs)

    cp_e2.wait()

    def e2_body(r, accs):
        return tuple(accs[c] + e2v[r, pl.ds(c * L, L)]
                     for c in range(DH // L))

    accs = lax.fori_loop(e2_lo - c2, e2_hi - c2, e2_body, accs)

    cp_tp.wait()
    accs = tuple(accs[c] + tpv[m, pl.ds(col0 + c * L, L)]
                 for c in range(DH // L))

    for c in range(DH // L):
        outv[pl.ds(c * L, L)] = accs[c] * inv
    pltpu.sync_copy(outv, slab.at[m % (B // NC), pl.ds(col0, DH)])
    plsc.subcore_barrier()

    @pl.when(sid == 0)
    def _():
        pltpu.sync_copy(slab, out_hbm.at[pl.ds(sc * (B // NC), B // NC)])


def kernel(atom_hiddens, a_scope):
    x = atom_hiddens.astype(jnp.float32)
    s32 = a_scope[:, 0].astype(jnp.int32)
    z32 = a_scope[:, 1].astype(jnp.int32)
    scope2 = jnp.stack([s32, z32])
    s_col = s32[:, None]
    e_col = jnp.minimum(s32 + z32, N)[:, None]
    bs_sc = _block_sums(x)
    tc_part = _tc_partials(s_col, e_col, x)
    return _combine(x, bs_sc, tc_part, scope2)
